# Initial kernel scaffold; baseline (speedup 1.0000x reference)
import jax
import jax.numpy as jnp
from jax.experimental import pallas as pl


def kernel(h_V, h_E, topology, params):
    raise NotImplementedError("write your pallas kernel here")


# speedup vs baseline: 9.0254x; 9.0254x over previous
import functools

import jax
import jax.numpy as jnp
from jax.experimental import pallas as pl
from jax.experimental.pallas import tpu as pltpu
from jax.experimental.pallas import tpu_sc as plsc

RS = 0.7071
SCALE = 60.0
LN_EPS = 1e-6

BN = 400
K = 32
C = 128


def _dot(a, b):
    return jnp.dot(a, b, preferred_element_type=jnp.float32)


def _ln(x, s, b):
    mu = jnp.mean(x, axis=-1, keepdims=True)
    var = jnp.mean((x - mu) ** 2, axis=-1, keepdims=True)
    return (x - mu) * jax.lax.rsqrt(var + LN_EPS) * s + b


def _proj_body(hv_ref, w_ref, out_ref):
    out_ref[...] = _dot(hv_ref[...], w_ref[...])


def _proj(hv, w):
    n = hv.shape[0]
    bn = 2000
    return pl.pallas_call(
        _proj_body,
        grid=(n // bn,),
        in_specs=[
            pl.BlockSpec((bn, C), lambda i: (i, 0)),
            pl.BlockSpec((C, C), lambda i: (0, 0)),
        ],
        out_specs=pl.BlockSpec((bn, C), lambda i: (i, 0)),
        out_shape=jax.ShapeDtypeStruct((n, C), jnp.float32),
    )(hv, w)


def _sc_gather(table, idx2d):
    e = idx2d.shape[1]
    win = 128
    mesh = plsc.VectorSubcoreMesh(core_axis_name="c", subcore_axis_name="s")

    @functools.partial(
        pl.kernel,
        out_type=jax.ShapeDtypeStruct((e, C), table.dtype),
        mesh=mesh,
    )
    def gather_kernel(x_hbm, i_hbm, o_hbm):
        def body(i_vmem, o_vmem):
            pltpu.sync_copy(x_hbm.at[i_vmem.at[0]], o_vmem)

        pltpu.emit_pipeline(
            body,
            grid=(e // win,),
            in_specs=[pl.BlockSpec((1, win), lambda i: (0, i))],
            out_specs=[pl.BlockSpec((win, C), lambda i: (i, 0))],
            core_axis_name=("c", "s"),
            dimension_semantics=(pltpu.PARALLEL,),
        )(i_hbm, o_hbm)

    return gather_kernel(table, idx2d)


def _node_body(hv_ref, he_ref, g1_ref,
               w0a, w0c, b0, w1, b1, w2, b2,
               f0, fb0, f1, fb1, s1, be1, s2, be2, u0b,
               hv2_ref, vp2_ref):
    hv = hv_ref[...]
    a = _dot(hv, w0a[...]) + b0[...]
    pre = g1_ref[...] + _dot(he_ref[...], w0c[...])
    pre = pre.reshape(BN, K, C) + a[:, None, :]
    h1 = jax.nn.gelu(pre.reshape(BN * K, C))
    h2 = jax.nn.gelu(_dot(h1, w1[...]) + b1[...])
    msg = _dot(h2, w2[...]) + b2[...]
    dh = msg.reshape(BN, K, C).sum(axis=1) * (1.0 / SCALE)
    v1 = _ln(RS * hv + dh, s1[...], be1[...])
    f = _dot(jax.nn.gelu(_dot(v1, f0[...]) + fb0[...]), f1[...]) + fb1[...]
    v2 = _ln(RS * v1 + f, s2[...], be2[...])
    hv2_ref[...] = v2
    vp2_ref[...] = _dot(v2, u0b[...])


def _edge_body(hv2_ref, he_ref, g2_ref,
               u0a, u0c, c0, u1, c1, u2, c2, s3, be3,
               out_ref):
    a = _dot(hv2_ref[...], u0a[...]) + c0[...]
    pre = g2_ref[...] + _dot(he_ref[...], u0c[...])
    pre = pre.reshape(BN, K, C) + a[:, None, :]
    h1 = jax.nn.gelu(pre.reshape(BN * K, C))
    h2 = jax.nn.gelu(_dot(h1, u1[...]) + c1[...])
    upd = _dot(h2, u2[...]) + c2[...]
    out_ref[...] = _ln(RS * he_ref[...] + upd, s3[...], be3[...])


def _mat_spec():
    return pl.BlockSpec((C, C), lambda i: (0, 0))


def _vec_spec():
    return pl.BlockSpec((1, C), lambda i: (0, 0))


def kernel(h_V, h_E, topology, params):
    B, N, Kk, Cc = h_E.shape
    E = N * Kk
    EB = BN * Kk

    hv = h_V[0]
    he = h_E[0].reshape(E, Cc)
    idx = topology[0].reshape(1, E).astype(jnp.int32)

    (W0, b0), (W1, b1), (W2, b2) = params["node_mlp"]
    (U0, c0), (U1, c1), (U2, c2) = params["edge_mlp"]
    (F0, fb0), (F1, fb1) = params["ffn"]
    s1, be1 = params["ln1"]
    s2, be2 = params["ln2"]
    s3, be3 = params["ln3"]

    W0a, W0b, W0c = W0[:Cc], W0[Cc:2 * Cc], W0[2 * Cc:]
    U0a, U0b, U0c = U0[:Cc], U0[Cc:2 * Cc], U0[2 * Cc:]
    row = lambda v: v.reshape(1, -1)

    vp1 = _proj(hv, W0b)
    g1 = _sc_gather(vp1, idx)

    node_grid = N // BN
    hv2, vp2 = pl.pallas_call(
        _node_body,
        grid=(node_grid,),
        in_specs=[
            pl.BlockSpec((BN, Cc), lambda i: (i, 0)),
            pl.BlockSpec((EB, Cc), lambda i: (i, 0)),
            pl.BlockSpec((EB, Cc), lambda i: (i, 0)),
            _mat_spec(), _mat_spec(), _vec_spec(),
            _mat_spec(), _vec_spec(),
            _mat_spec(), _vec_spec(),
            _mat_spec(), _vec_spec(),
            _mat_spec(), _vec_spec(),
            _vec_spec(), _vec_spec(),
            _vec_spec(), _vec_spec(),
            _mat_spec(),
        ],
        out_specs=[
            pl.BlockSpec((BN, Cc), lambda i: (i, 0)),
            pl.BlockSpec((BN, Cc), lambda i: (i, 0)),
        ],
        out_shape=[
            jax.ShapeDtypeStruct((N, Cc), jnp.float32),
            jax.ShapeDtypeStruct((N, Cc), jnp.float32),
        ],
    )(hv, he, g1,
      W0a, W0c, row(b0), W1, row(b1), W2, row(b2),
      F0, row(fb0), F1, row(fb1), row(s1), row(be1), row(s2), row(be2),
      U0b)

    g2 = _sc_gather(vp2, idx)

    he2 = pl.pallas_call(
        _edge_body,
        grid=(node_grid,),
        in_specs=[
            pl.BlockSpec((BN, Cc), lambda i: (i, 0)),
            pl.BlockSpec((EB, Cc), lambda i: (i, 0)),
            pl.BlockSpec((EB, Cc), lambda i: (i, 0)),
            _mat_spec(), _mat_spec(), _vec_spec(),
            _mat_spec(), _vec_spec(),
            _mat_spec(), _vec_spec(),
            _vec_spec(), _vec_spec(),
        ],
        out_specs=pl.BlockSpec((EB, Cc), lambda i: (i, 0)),
        out_shape=jax.ShapeDtypeStruct((E, Cc), jnp.float32),
    )(hv2, he, g2,
      U0a, U0c, row(c0), U1, row(c1), U2, row(c2), row(s3), row(be3))

    return hv2.reshape(B, N, Cc), he2.reshape(B, N, Kk, Cc)


# speedup vs baseline: 9.8095x; 1.0869x over previous
import jax
import jax.numpy as jnp
from jax.experimental import pallas as pl
from jax.experimental.pallas import tpu as pltpu
from jax.experimental.pallas import tpu_sc as plsc

RS = 0.7071
SCALE = 60.0
LN_EPS = 1e-6

K = 32
C = 128
BN = 400
CHUNKS = 5


def _dot(a, b):
    return jnp.dot(a, b, preferred_element_type=jnp.float32)


def _ln(x, s, b):
    mu = jnp.mean(x, axis=-1, keepdims=True)
    var = jnp.mean((x - mu) ** 2, axis=-1, keepdims=True)
    return (x - mu) * jax.lax.rsqrt(var + LN_EPS) * s + b


def _proj_body(hv_ref, w_ref, out_ref):
    out_ref[...] = _dot(hv_ref[...], w_ref[...])


def _proj(hv, w):
    n = hv.shape[0]
    bn = 2000
    return pl.pallas_call(
        _proj_body,
        grid=(n // bn,),
        in_specs=[
            pl.BlockSpec((bn, C), lambda i: (i, 0)),
            pl.BlockSpec((C, C), lambda i: (0, 0)),
        ],
        out_specs=pl.BlockSpec((bn, C), lambda i: (i, 0)),
        out_shape=jax.ShapeDtypeStruct((n, C), jnp.float32),
    )(hv, w)


def _sc_gather(table, idx2d):
    e = idx2d.shape[1]
    win = 128
    mesh = plsc.VectorSubcoreMesh(core_axis_name="c", subcore_axis_name="s")

    def gather_kernel(x_hbm, i_hbm, o_hbm):
        def body(i_vmem, o_vmem):
            pltpu.sync_copy(x_hbm.at[i_vmem.at[0]], o_vmem)

        pltpu.emit_pipeline(
            body,
            grid=(e // win,),
            in_specs=[pl.BlockSpec((1, win), lambda i: (0, i))],
            out_specs=[pl.BlockSpec((win, C), lambda i: (i, 0))],
            core_axis_name=("c", "s"),
            dimension_semantics=(pltpu.PARALLEL,),
        )(i_hbm, o_hbm)

    return pl.kernel(
        gather_kernel,
        out_type=jax.ShapeDtypeStruct((e, C), table.dtype),
        mesh=mesh,
    )(table, idx2d)


def _node_body(hv_ref, he_ref, g1_ref,
               w0a, w0c, b0, w1, b1, w2, b2,
               f0, fb0, f1, fb1, s1, be1, s2, be2, u0b,
               hv2_ref, vp2_ref):
    hv = hv_ref[...]
    a = _dot(hv, w0a[...]) + b0[...]
    pre = g1_ref[...] + _dot(he_ref[...], w0c[...])
    pre = pre.reshape(BN, K, C) + a[:, None, :]
    h1 = jax.nn.gelu(pre.reshape(BN * K, C))
    h2 = jax.nn.gelu(_dot(h1, w1[...]) + b1[...])
    msg = _dot(h2, w2[...]) + b2[...]
    dh = msg.reshape(BN, K, C).sum(axis=1) * (1.0 / SCALE)
    v1 = _ln(RS * hv + dh, s1[...], be1[...])
    f = _dot(jax.nn.gelu(_dot(v1, f0[...]) + fb0[...]), f1[...]) + fb1[...]
    v2 = _ln(RS * v1 + f, s2[...], be2[...])
    hv2_ref[...] = v2
    vp2_ref[...] = _dot(v2, u0b[...])


def _edge_body(hv2_ref, he_ref, g2_ref,
               u0a, u0c, c0, u1, c1, u2, c2, s3, be3,
               out_ref):
    a = _dot(hv2_ref[...], u0a[...]) + c0[...]
    pre = g2_ref[...] + _dot(he_ref[...], u0c[...])
    pre = pre.reshape(BN, K, C) + a[:, None, :]
    h1 = jax.nn.gelu(pre.reshape(BN * K, C))
    h2 = jax.nn.gelu(_dot(h1, u1[...]) + c1[...])
    upd = _dot(h2, u2[...]) + c2[...]
    out_ref[...] = _ln(RS * he_ref[...] + upd, s3[...], be3[...])


def _edge_body_aliased(buf_ref, hv2_ref, he_ref, g2_ref,
                       u0a, u0c, c0, u1, c1, u2, c2, s3, be3,
                       out_ref):
    del buf_ref
    _edge_body(hv2_ref, he_ref, g2_ref,
               u0a, u0c, c0, u1, c1, u2, c2, s3, be3, out_ref)


def _mat_spec():
    return pl.BlockSpec((C, C), lambda i: (0, 0))


def _vec_spec():
    return pl.BlockSpec((1, C), lambda i: (0, 0))


def kernel(h_V, h_E, topology, params):
    B, N, Kk, Cc = h_E.shape
    E = N * Kk
    EB = BN * Kk
    CN = N // CHUNKS
    CB = CN // BN
    CE = CN * Kk

    hv = h_V[0]
    he = h_E[0].reshape(E, Cc)
    idx = topology[0].reshape(1, E).astype(jnp.int32)

    (W0, b0), (W1, b1), (W2, b2) = params["node_mlp"]
    (U0, c0), (U1, c1), (U2, c2) = params["edge_mlp"]
    (F0, fb0), (F1, fb1) = params["ffn"]
    s1, be1 = params["ln1"]
    s2, be2 = params["ln2"]
    s3, be3 = params["ln3"]

    W0a, W0b, W0c = W0[:Cc], W0[Cc:2 * Cc], W0[2 * Cc:]
    U0a, U0b, U0c = U0[:Cc], U0[Cc:2 * Cc], U0[2 * Cc:]
    row = lambda v: v.reshape(1, -1)

    node_w = (W0a, W0c, row(b0), W1, row(b1), W2, row(b2),
              F0, row(fb0), F1, row(fb1), row(s1), row(be1), row(s2), row(be2),
              U0b)
    node_w_specs = [
        _mat_spec(), _mat_spec(), _vec_spec(),
        _mat_spec(), _vec_spec(),
        _mat_spec(), _vec_spec(),
        _mat_spec(), _vec_spec(),
        _mat_spec(), _vec_spec(),
        _vec_spec(), _vec_spec(),
        _vec_spec(), _vec_spec(),
        _mat_spec(),
    ]
    edge_w = (U0a, U0c, row(c0), U1, row(c1), U2, row(c2), row(s3), row(be3))
    edge_w_specs = [
        _mat_spec(), _mat_spec(), _vec_spec(),
        _mat_spec(), _vec_spec(),
        _mat_spec(), _vec_spec(),
        _vec_spec(), _vec_spec(),
    ]

    vp1 = _proj(hv, W0b)

    idx_chunks = [jax.lax.slice(idx, (0, c * CE), (1, (c + 1) * CE))
                  for c in range(CHUNKS)]

    g1 = [_sc_gather(vp1, idx_chunks[c]) for c in range(CHUNKS)]

    hv2_c, vp2_c = [], []
    for c in range(CHUNKS):
        off = c * CB
        h2c, v2c = pl.pallas_call(
            _node_body,
            grid=(CB,),
            in_specs=[
                pl.BlockSpec((BN, Cc), lambda i, off=off: (i + off, 0)),
                pl.BlockSpec((EB, Cc), lambda i, off=off: (i + off, 0)),
                pl.BlockSpec((EB, Cc), lambda i: (i, 0)),
                *node_w_specs,
            ],
            out_specs=[
                pl.BlockSpec((BN, Cc), lambda i: (i, 0)),
                pl.BlockSpec((BN, Cc), lambda i: (i, 0)),
            ],
            out_shape=[
                jax.ShapeDtypeStruct((CN, Cc), jnp.float32),
                jax.ShapeDtypeStruct((CN, Cc), jnp.float32),
            ],
        )(hv, he, g1[c], *node_w)
        hv2_c.append(h2c)
        vp2_c.append(v2c)

    hv2 = jnp.concatenate(hv2_c, axis=0)
    vp2 = jnp.concatenate(vp2_c, axis=0)

    g2 = [_sc_gather(vp2, idx_chunks[c]) for c in range(CHUNKS)]

    he2 = None
    for c in range(CHUNKS):
        off = c * CB
        data_specs = [
            pl.BlockSpec((BN, Cc), lambda i, off=off: (i + off, 0)),
            pl.BlockSpec((EB, Cc), lambda i, off=off: (i + off, 0)),
            pl.BlockSpec((EB, Cc), lambda i: (i, 0)),
        ]
        out_spec = pl.BlockSpec((EB, Cc), lambda i, off=off: (i + off, 0))
        out_shape = jax.ShapeDtypeStruct((E, Cc), jnp.float32)
        if c == 0:
            he2 = pl.pallas_call(
                _edge_body,
                grid=(CB,),
                in_specs=data_specs + edge_w_specs,
                out_specs=out_spec,
                out_shape=out_shape,
            )(hv2, he, g2[c], *edge_w)
        else:
            he2 = pl.pallas_call(
                _edge_body_aliased,
                grid=(CB,),
                in_specs=[pl.BlockSpec((8, Cc), lambda i: (0, 0))]
                + data_specs + edge_w_specs,
                out_specs=out_spec,
                out_shape=out_shape,
                input_output_aliases={0: 0},
            )(he2, hv2, he, g2[c], *edge_w)

    return hv2.reshape(B, N, Cc), he2.reshape(B, N, Kk, Cc)


# speedup vs baseline: 10.5573x; 1.0762x over previous
import jax
import jax.numpy as jnp
from jax.experimental import pallas as pl
from jax.experimental.pallas import tpu as pltpu
from jax.experimental.pallas import tpu_sc as plsc

RS = 0.7071
SCALE = 60.0
LN_EPS = 1e-6

K = 32
C = 128
BN = 400
CHUNKS = 5


def _dot(a, b):
    return jnp.dot(a, b, preferred_element_type=jnp.float32)


def _ln(x, s, b):
    mu = jnp.mean(x, axis=-1, keepdims=True)
    var = jnp.mean((x - mu) ** 2, axis=-1, keepdims=True)
    return (x - mu) * jax.lax.rsqrt(var + LN_EPS) * s + b


def _proj_body(hv_ref, w_ref, out_ref):
    out_ref[...] = _dot(hv_ref[...], w_ref[...])


def _proj(hv, w):
    n = hv.shape[0]
    bn = 2000
    return pl.pallas_call(
        _proj_body,
        grid=(n // bn,),
        in_specs=[
            pl.BlockSpec((bn, C), lambda i: (i, 0)),
            pl.BlockSpec((C, C), lambda i: (0, 0)),
        ],
        out_specs=pl.BlockSpec((bn, C), lambda i: (i, 0)),
        out_shape=jax.ShapeDtypeStruct((n, C), jnp.float32),
    )(hv, w)


def _sc_gather(table, idx2d):
    e = idx2d.shape[1]
    win = 128
    mesh = plsc.VectorSubcoreMesh(core_axis_name="c", subcore_axis_name="s")

    def gather_kernel(x_hbm, i_hbm, o_hbm):
        def body(i_vmem, o_vmem):
            pltpu.sync_copy(x_hbm.at[i_vmem.at[0]], o_vmem)

        pltpu.emit_pipeline(
            body,
            grid=(e // win,),
            in_specs=[pl.BlockSpec((1, win), lambda i: (0, i))],
            out_specs=[pl.BlockSpec((win, C), lambda i: (i, 0))],
            core_axis_name=("c", "s"),
            dimension_semantics=(pltpu.PARALLEL,),
        )(i_hbm, o_hbm)

    return pl.kernel(
        gather_kernel,
        out_type=jax.ShapeDtypeStruct((e, C), table.dtype),
        mesh=mesh,
    )(table, idx2d)


def _bf(x):
    return x.astype(jnp.bfloat16)


def _msg_mlp(a, g, he, w0c, w1, b1, w2, b2):
    pre = g + _dot(_bf(he), _bf(w0c))
    pre = _bf(pre).reshape(BN, K, C) + _bf(a)[:, None, :]
    h1 = jax.nn.gelu(pre.reshape(BN * K, C))
    h2 = jax.nn.gelu(_bf(_dot(h1, _bf(w1)) + b1))
    return _dot(h2, _bf(w2))


def _node_body(hv_ref, he_ref, g1_ref,
               w0a, w0c, b0, w1, b1, w2, b2,
               f0, fb0, f1, fb1, s1, be1, s2, be2, u0b,
               hv2_ref, vp2_ref):
    hv = hv_ref[...]
    a = _dot(hv, w0a[...]) + b0[...]
    msg = _msg_mlp(a, g1_ref[...], he_ref[...],
                   w0c[...], w1[...], b1[...], w2[...], b2[...])
    dh = (msg.reshape(BN, K, C).sum(axis=1) + K * b2[...]) * (1.0 / SCALE)
    v1 = _ln(RS * hv + dh, s1[...], be1[...])
    f = _dot(_bf(jax.nn.gelu(_dot(v1, f0[...]) + fb0[...])), _bf(f1[...])) \
        + fb1[...]
    v2 = _ln(RS * v1 + f, s2[...], be2[...])
    hv2_ref[...] = v2
    vp2_ref[...] = _dot(v2, u0b[...])


def _edge_body(hv2_ref, he_ref, g2_ref,
               u0a, u0c, c0, u1, c1, u2, c2, s3, be3,
               out_ref):
    a = _dot(hv2_ref[...], u0a[...]) + c0[...]
    upd = _msg_mlp(a, g2_ref[...], he_ref[...],
                   u0c[...], u1[...], c1[...], u2[...], c2[...])
    out_ref[...] = _ln(RS * he_ref[...] + upd + c2[...],
                       s3[...], be3[...])


def _edge_body_aliased(buf_ref, hv2_ref, he_ref, g2_ref,
                       u0a, u0c, c0, u1, c1, u2, c2, s3, be3,
                       out_ref):
    del buf_ref
    _edge_body(hv2_ref, he_ref, g2_ref,
               u0a, u0c, c0, u1, c1, u2, c2, s3, be3, out_ref)


def _mat_spec():
    return pl.BlockSpec((C, C), lambda i: (0, 0))


def _vec_spec():
    return pl.BlockSpec((1, C), lambda i: (0, 0))


def kernel(h_V, h_E, topology, params):
    B, N, Kk, Cc = h_E.shape
    E = N * Kk
    EB = BN * Kk
    CN = N // CHUNKS
    CB = CN // BN
    CE = CN * Kk

    hv = h_V[0]
    he = h_E[0].reshape(E, Cc)
    idx = topology[0].reshape(1, E).astype(jnp.int32)

    (W0, b0), (W1, b1), (W2, b2) = params["node_mlp"]
    (U0, c0), (U1, c1), (U2, c2) = params["edge_mlp"]
    (F0, fb0), (F1, fb1) = params["ffn"]
    s1, be1 = params["ln1"]
    s2, be2 = params["ln2"]
    s3, be3 = params["ln3"]

    W0a, W0b, W0c = W0[:Cc], W0[Cc:2 * Cc], W0[2 * Cc:]
    U0a, U0b, U0c = U0[:Cc], U0[Cc:2 * Cc], U0[2 * Cc:]
    row = lambda v: v.reshape(1, -1)

    node_w = (W0a, W0c, row(b0), W1, row(b1), W2, row(b2),
              F0, row(fb0), F1, row(fb1), row(s1), row(be1), row(s2), row(be2),
              U0b)
    node_w_specs = [
        _mat_spec(), _mat_spec(), _vec_spec(),
        _mat_spec(), _vec_spec(),
        _mat_spec(), _vec_spec(),
        _mat_spec(), _vec_spec(),
        _mat_spec(), _vec_spec(),
        _vec_spec(), _vec_spec(),
        _vec_spec(), _vec_spec(),
        _mat_spec(),
    ]
    edge_w = (U0a, U0c, row(c0), U1, row(c1), U2, row(c2), row(s3), row(be3))
    edge_w_specs = [
        _mat_spec(), _mat_spec(), _vec_spec(),
        _mat_spec(), _vec_spec(),
        _mat_spec(), _vec_spec(),
        _vec_spec(), _vec_spec(),
    ]

    vp1 = _proj(hv, W0b)

    idx_chunks = [jax.lax.slice(idx, (0, c * CE), (1, (c + 1) * CE))
                  for c in range(CHUNKS)]

    g1 = [_sc_gather(vp1, idx_chunks[c]) for c in range(CHUNKS)]

    hv2_c, vp2_c = [], []
    for c in range(CHUNKS):
        off = c * CB
        h2c, v2c = pl.pallas_call(
            _node_body,
            grid=(CB,),
            in_specs=[
                pl.BlockSpec((BN, Cc), lambda i, off=off: (i + off, 0)),
                pl.BlockSpec((EB, Cc), lambda i, off=off: (i + off, 0)),
                pl.BlockSpec((EB, Cc), lambda i: (i, 0)),
                *node_w_specs,
            ],
            out_specs=[
                pl.BlockSpec((BN, Cc), lambda i: (i, 0)),
                pl.BlockSpec((BN, Cc), lambda i: (i, 0)),
            ],
            out_shape=[
                jax.ShapeDtypeStruct((CN, Cc), jnp.float32),
                jax.ShapeDtypeStruct((CN, Cc), jnp.float32),
            ],
        )(hv, he, g1[c], *node_w)
        hv2_c.append(h2c)
        vp2_c.append(v2c)

    hv2 = jnp.concatenate(hv2_c, axis=0)
    vp2 = jnp.concatenate(vp2_c, axis=0)

    g2 = [_sc_gather(vp2, idx_chunks[c]) for c in range(CHUNKS)]

    he2 = None
    for c in range(CHUNKS):
        off = c * CB
        data_specs = [
            pl.BlockSpec((BN, Cc), lambda i, off=off: (i + off, 0)),
            pl.BlockSpec((EB, Cc), lambda i, off=off: (i + off, 0)),
            pl.BlockSpec((EB, Cc), lambda i: (i, 0)),
        ]
        out_spec = pl.BlockSpec((EB, Cc), lambda i, off=off: (i + off, 0))
        out_shape = jax.ShapeDtypeStruct((E, Cc), jnp.float32)
        if c == 0:
            he2 = pl.pallas_call(
                _edge_body,
                grid=(CB,),
                in_specs=data_specs + edge_w_specs,
                out_specs=out_spec,
                out_shape=out_shape,
            )(hv2, he, g2[c], *edge_w)
        else:
            he2 = pl.pallas_call(
                _edge_body_aliased,
                grid=(CB,),
                in_specs=[pl.BlockSpec((8, Cc), lambda i: (0, 0))]
                + data_specs + edge_w_specs,
                out_specs=out_spec,
                out_shape=out_shape,
                input_output_aliases={0: 0},
            )(he2, hv2, he, g2[c], *edge_w)

    return hv2.reshape(B, N, Cc), he2.reshape(B, N, Kk, Cc)
